# baseline (speedup 1.0000x reference)
import functools

import jax
import jax.numpy as jnp
from jax import lax
from jax.experimental import pallas as pl
from jax.experimental.pallas import tpu as pltpu
from jax.experimental.pallas import tpu_sc as plsc

N = 10000
E = 320000
D = 128
G = 64

NC = 2
NS = 16
NW = NC * NS
EPW = E // NW
C = 80
CH = EPW // C
KG = 4
NB = 2 * KG
NG = CH // (2 * KG)
ADT = jnp.bfloat16
RPT = 624
TAIL = N - NS * RPT


def _make_agg():
    mesh = plsc.VectorSubcoreMesh(core_axis_name="c", subcore_axis_name="s")

    @functools.partial(
        pl.kernel,
        out_type=jax.ShapeDtypeStruct((NC, N, D), ADT),
        mesh=mesh,
        compiler_params=pltpu.CompilerParams(use_tc_tiling_on_sc=False,
                                             skip_device_barrier=True),
        scratch_types=[
            pltpu.VMEM((CH, C), jnp.int32),
            pltpu.VMEM((CH, C), jnp.int32),
            [pltpu.VMEM((C, D), ADT) for _ in range(NB)],
            pltpu.VMEM_SHARED((N, D), ADT),
            [pltpu.SemaphoreType.DMA for _ in range(2)],
            [pltpu.SemaphoreType.DMA for _ in range(2)],
        ],
    )
    def agg(h_hbm, src_hbm, dst_hbm, zeros_hbm, out_hbm,
            src_v, dst_v, rows, acc, gsem, ssem):
        cid = lax.axis_index("c")
        sid = lax.axis_index("s")
        wid = cid * NS + sid
        pltpu.sync_copy(src_hbm.at[wid], src_v)
        pltpu.sync_copy(dst_hbm.at[wid], dst_v)

        def g_start(p, b, j):
            pltpu.async_copy(h_hbm.at[src_v.at[j]], rows[p * KG + b], gsem[p])

        def g_wait(p, b, j):
            pltpu.make_async_copy(h_hbm.at[src_v.at[j]], rows[p * KG + b],
                                  gsem[p]).wait()

        def s_start(p, b, j):
            pltpu.async_copy(rows[p * KG + b], acc.at[dst_v.at[j]], ssem[p],
                             add=True)

        def s_wait(p, b, j):
            pltpu.make_async_copy(rows[p * KG + b], acc.at[dst_v.at[j]],
                                  ssem[p]).wait()

        for p in range(2):
            for b in range(KG):
                g_start(p, b, p * KG + b)

        row0 = pl.multiple_of(sid * RPT, 8)
        pltpu.sync_copy(zeros_hbm.at[pl.ds(row0, RPT)],
                        acc.at[pl.ds(row0, RPT)])

        @pl.when(sid == NS - 1)
        def _():
            pltpu.sync_copy(zeros_hbm.at[pl.ds(NS * RPT, TAIL)],
                            acc.at[pl.ds(NS * RPT, TAIL)])

        plsc.subcore_barrier()

        def step(g, carry):
            base = g * 2 * KG
            for p in range(2):
                jp = base + p * KG
                for b in range(KG):
                    g_wait(p, b, jp + b)
                for b in range(KG):
                    s_start(p, b, jp + b)
                for b in range(KG):
                    s_wait(p, b, jp + b)
                for b in range(KG):
                    nxt = jp + b + 2 * KG

                    @pl.when(nxt < CH)
                    def _(p=p, b=b, nxt=nxt):
                        g_start(p, b, nxt)
            return carry

        lax.fori_loop(0, NG, step, 0)
        tail = [(j, (j % (2 * KG)) // KG, j % KG)
                for j in range(NG * 2 * KG, CH)]
        for j, p, b in tail:
            g_wait(p, b, j)
        for j, p, b in tail:
            s_start(p, b, j)
        for j, p, b in tail:
            s_wait(p, b, j)
        plsc.subcore_barrier()
        pltpu.sync_copy(acc.at[pl.ds(row0, RPT)],
                        out_hbm.at[cid, pl.ds(row0, RPT)])

        @pl.when(sid == NS - 1)
        def _():
            pltpu.sync_copy(acc.at[pl.ds(NS * RPT, TAIL)],
                            out_hbm.at[cid, pl.ds(NS * RPT, TAIL)])

    return agg


_agg = _make_agg()


_RB = 2000


def _mlp_body(s_ref, h_ref, a0_ref, a1_ref, w1_ref, b1_ref, w2_ref, b2_ref,
              ob_ref):
    z = (h_ref[...].astype(jnp.float32) * s_ref[0]
         + a0_ref[...].astype(jnp.float32)
         + a1_ref[...].astype(jnp.float32))
    z = jnp.maximum(
        jnp.dot(z, w1_ref[...], preferred_element_type=jnp.float32)
        + b1_ref[...], 0.0)
    h2 = jnp.maximum(
        jnp.dot(z, w2_ref[...], preferred_element_type=jnp.float32)
        + b2_ref[...], 0.0)
    ob_ref[...] = h2.astype(ADT)


def _gin_mlp(scale, h, a0, a1, w1, b1, w2, b2):
    nb = N // _RB
    return pl.pallas_call(
        _mlp_body,
        grid=(nb,),
        in_specs=[
            pl.BlockSpec(memory_space=pltpu.SMEM),
            pl.BlockSpec((_RB, D), lambda i: (i, 0)),
            pl.BlockSpec((_RB, D), lambda i: (i, 0)),
            pl.BlockSpec((_RB, D), lambda i: (i, 0)),
            pl.BlockSpec((D, D), lambda i: (0, 0)),
            pl.BlockSpec((1, D), lambda i: (0, 0)),
            pl.BlockSpec((D, D), lambda i: (0, 0)),
            pl.BlockSpec((1, D), lambda i: (0, 0)),
        ],
        out_specs=pl.BlockSpec((_RB, D), lambda i: (i, 0)),
        out_shape=jax.ShapeDtypeStruct((N, D), ADT),
    )(scale, h, a0, a1, w1, b1, w2, b2)


def _head_body(h_ref, batch_ref, wn_ref, bn_ref, wp_ref, bp_ref, o_ref,
               sums, cnts):
    i = pl.program_id(0)
    nb = pl.num_programs(0)
    hn = jnp.maximum(
        jnp.dot(h_ref[...].astype(jnp.float32), wn_ref[...],
                preferred_element_type=jnp.float32)
        + bn_ref[...], 0.0)
    onehot = (batch_ref[...] ==
              lax.broadcasted_iota(jnp.int32, (_RB, G), 1)).astype(jnp.float32)
    dn = (((0,), (0,)), ((), ()))
    psum = lax.dot_general(onehot, hn, dn,
                           preferred_element_type=jnp.float32)
    pcnt = lax.dot_general(onehot, jnp.ones((_RB, D), jnp.float32), dn,
                           preferred_element_type=jnp.float32)

    @pl.when(i == 0)
    def _():
        sums[...] = psum
        cnts[...] = pcnt

    @pl.when(i > 0)
    def _():
        sums[...] += psum
        cnts[...] += pcnt

    @pl.when(i == nb - 1)
    def _():
        mean = sums[...] / jnp.maximum(cnts[...], 1.0)
        o_ref[...] = (jnp.dot(mean, wp_ref[...],
                              preferred_element_type=jnp.float32)
                      + bp_ref[...])


def _head(h, batch2, wn, bn, wp, bp):
    nb = N // _RB
    t = wp.shape[1]
    return pl.pallas_call(
        _head_body,
        grid=(nb,),
        in_specs=[
            pl.BlockSpec((_RB, D), lambda i: (i, 0)),
            pl.BlockSpec((_RB, 1), lambda i: (i, 0)),
            pl.BlockSpec((D, D), lambda i: (0, 0)),
            pl.BlockSpec((1, D), lambda i: (0, 0)),
            pl.BlockSpec((D, t), lambda i: (0, 0)),
            pl.BlockSpec((1, t), lambda i: (0, 0)),
        ],
        out_specs=pl.BlockSpec((G, t), lambda i: (0, 0)),
        out_shape=jax.ShapeDtypeStruct((G, t), jnp.float32),
        scratch_shapes=[
            pltpu.VMEM((G, D), jnp.float32),
            pltpu.VMEM((G, D), jnp.float32),
        ],
    )(h, batch2, wn, bn, wp, bp)


def kernel(x, edge_index, batch, W1, b1, W2, b2, eps, Wn, bn, Wp, bp):
    n_layers = W1.shape[0]
    t = Wp.shape[1]
    src = edge_index[0].astype(jnp.int32).reshape(NW, CH, C)
    dst = edge_index[1].astype(jnp.int32).reshape(NW, CH, C)
    batch2 = batch.astype(jnp.int32).reshape(N, 1)
    zeros = jnp.zeros((N, D), ADT)

    hb = x.astype(ADT)
    for i in range(n_layers):
        parts = _agg(hb, src, dst, zeros)
        scale = (1.0 + eps[i]).reshape(1).astype(jnp.float32)
        hb = _gin_mlp(scale, hb, parts[0], parts[1], W1[i],
                      b1[i].reshape(1, D), W2[i], b2[i].reshape(1, D))
    return _head(hb, batch2, Wn, bn.reshape(1, D), Wp, bp.reshape(1, t))


# speedup vs baseline: 1.0004x; 1.0004x over previous
import functools

import jax
import jax.numpy as jnp
from jax import lax
from jax.experimental import pallas as pl
from jax.experimental.pallas import tpu as pltpu
from jax.experimental.pallas import tpu_sc as plsc

N = 10000
E = 320000
D = 128
G = 64

NC = 2
NS = 16
NW = NC * NS
EPW = E // NW
C = 80
CH = EPW // C
KG = 4
NB = 2 * KG
NG = CH // (2 * KG)
ADT = jnp.bfloat16
RPT = 624
TAIL = N - NS * RPT


def _make_agg():
    mesh = plsc.VectorSubcoreMesh(core_axis_name="c", subcore_axis_name="s")

    @functools.partial(
        pl.kernel,
        out_type=jax.ShapeDtypeStruct((NC, N, D), ADT),
        mesh=mesh,
        compiler_params=pltpu.CompilerParams(use_tc_tiling_on_sc=False),
        scratch_types=[
            pltpu.VMEM((CH, C), jnp.int32),
            pltpu.VMEM((CH, C), jnp.int32),
            [pltpu.VMEM((C, D), ADT) for _ in range(NB)],
            pltpu.VMEM_SHARED((N, D), ADT),
            [pltpu.SemaphoreType.DMA for _ in range(2)],
            [pltpu.SemaphoreType.DMA for _ in range(2)],
        ],
    )
    def agg(h_hbm, src_hbm, dst_hbm, zeros_hbm, out_hbm,
            src_v, dst_v, rows, acc, gsem, ssem):
        cid = lax.axis_index("c")
        sid = lax.axis_index("s")
        wid = cid * NS + sid
        pltpu.sync_copy(src_hbm.at[wid], src_v)
        pltpu.sync_copy(dst_hbm.at[wid], dst_v)

        def g_start(p, b, j):
            pltpu.async_copy(h_hbm.at[src_v.at[j]], rows[p * KG + b], gsem[p])

        def g_wait(p, b, j):
            pltpu.make_async_copy(h_hbm.at[src_v.at[j]], rows[p * KG + b],
                                  gsem[p]).wait()

        def s_start(p, b, j):
            pltpu.async_copy(rows[p * KG + b], acc.at[dst_v.at[j]], ssem[p],
                             add=True)

        def s_wait(p, b, j):
            pltpu.make_async_copy(rows[p * KG + b], acc.at[dst_v.at[j]],
                                  ssem[p]).wait()

        for p in range(2):
            for b in range(KG):
                g_start(p, b, p * KG + b)

        row0 = pl.multiple_of(sid * RPT, 8)
        pltpu.sync_copy(zeros_hbm.at[pl.ds(row0, RPT)],
                        acc.at[pl.ds(row0, RPT)])

        @pl.when(sid == NS - 1)
        def _():
            pltpu.sync_copy(zeros_hbm.at[pl.ds(NS * RPT, TAIL)],
                            acc.at[pl.ds(NS * RPT, TAIL)])

        plsc.subcore_barrier()

        def step(g, carry):
            base = g * 2 * KG
            for p in range(2):
                jp = base + p * KG
                for b in range(KG):
                    g_wait(p, b, jp + b)
                for b in range(KG):
                    s_start(p, b, jp + b)
                for b in range(KG):
                    s_wait(p, b, jp + b)
                for b in range(KG):
                    nxt = jp + b + 2 * KG

                    @pl.when(nxt < CH)
                    def _(p=p, b=b, nxt=nxt):
                        g_start(p, b, nxt)
            return carry

        lax.fori_loop(0, NG, step, 0)
        tail = [(j, (j % (2 * KG)) // KG, j % KG)
                for j in range(NG * 2 * KG, CH)]
        for j, p, b in tail:
            g_wait(p, b, j)
        for j, p, b in tail:
            s_start(p, b, j)
        for j, p, b in tail:
            s_wait(p, b, j)
        plsc.subcore_barrier()
        pltpu.sync_copy(acc.at[pl.ds(row0, RPT)],
                        out_hbm.at[cid, pl.ds(row0, RPT)])

        @pl.when(sid == NS - 1)
        def _():
            pltpu.sync_copy(acc.at[pl.ds(NS * RPT, TAIL)],
                            out_hbm.at[cid, pl.ds(NS * RPT, TAIL)])

    return agg


_agg = _make_agg()


_RB = 2000


def _mlp_body(s_ref, h_ref, a0_ref, a1_ref, w1_ref, b1_ref, w2_ref, b2_ref,
              ob_ref):
    z = (h_ref[...].astype(jnp.float32) * s_ref[0]
         + a0_ref[...].astype(jnp.float32)
         + a1_ref[...].astype(jnp.float32))
    z = jnp.maximum(
        jnp.dot(z, w1_ref[...], preferred_element_type=jnp.float32)
        + b1_ref[...], 0.0)
    h2 = jnp.maximum(
        jnp.dot(z, w2_ref[...], preferred_element_type=jnp.float32)
        + b2_ref[...], 0.0)
    ob_ref[...] = h2.astype(ADT)


def _gin_mlp(scale, h, a0, a1, w1, b1, w2, b2):
    nb = N // _RB
    return pl.pallas_call(
        _mlp_body,
        grid=(nb,),
        in_specs=[
            pl.BlockSpec(memory_space=pltpu.SMEM),
            pl.BlockSpec((_RB, D), lambda i: (i, 0)),
            pl.BlockSpec((_RB, D), lambda i: (i, 0)),
            pl.BlockSpec((_RB, D), lambda i: (i, 0)),
            pl.BlockSpec((D, D), lambda i: (0, 0)),
            pl.BlockSpec((1, D), lambda i: (0, 0)),
            pl.BlockSpec((D, D), lambda i: (0, 0)),
            pl.BlockSpec((1, D), lambda i: (0, 0)),
        ],
        out_specs=pl.BlockSpec((_RB, D), lambda i: (i, 0)),
        out_shape=jax.ShapeDtypeStruct((N, D), ADT),
    )(scale, h, a0, a1, w1, b1, w2, b2)


def _head_body(h_ref, batch_ref, wn_ref, bn_ref, wp_ref, bp_ref, o_ref,
               sums, cnts):
    i = pl.program_id(0)
    nb = pl.num_programs(0)
    hn = jnp.maximum(
        jnp.dot(h_ref[...].astype(jnp.float32), wn_ref[...],
                preferred_element_type=jnp.float32)
        + bn_ref[...], 0.0)
    onehot = (batch_ref[...] ==
              lax.broadcasted_iota(jnp.int32, (_RB, G), 1)).astype(jnp.float32)
    dn = (((0,), (0,)), ((), ()))
    psum = lax.dot_general(onehot, hn, dn,
                           preferred_element_type=jnp.float32)
    pcnt = lax.dot_general(onehot, jnp.ones((_RB, D), jnp.float32), dn,
                           preferred_element_type=jnp.float32)

    @pl.when(i == 0)
    def _():
        sums[...] = psum
        cnts[...] = pcnt

    @pl.when(i > 0)
    def _():
        sums[...] += psum
        cnts[...] += pcnt

    @pl.when(i == nb - 1)
    def _():
        mean = sums[...] / jnp.maximum(cnts[...], 1.0)
        o_ref[...] = (jnp.dot(mean, wp_ref[...],
                              preferred_element_type=jnp.float32)
                      + bp_ref[...])


def _head(h, batch2, wn, bn, wp, bp):
    nb = N // _RB
    t = wp.shape[1]
    return pl.pallas_call(
        _head_body,
        grid=(nb,),
        in_specs=[
            pl.BlockSpec((_RB, D), lambda i: (i, 0)),
            pl.BlockSpec((_RB, 1), lambda i: (i, 0)),
            pl.BlockSpec((D, D), lambda i: (0, 0)),
            pl.BlockSpec((1, D), lambda i: (0, 0)),
            pl.BlockSpec((D, t), lambda i: (0, 0)),
            pl.BlockSpec((1, t), lambda i: (0, 0)),
        ],
        out_specs=pl.BlockSpec((G, t), lambda i: (0, 0)),
        out_shape=jax.ShapeDtypeStruct((G, t), jnp.float32),
        scratch_shapes=[
            pltpu.VMEM((G, D), jnp.float32),
            pltpu.VMEM((G, D), jnp.float32),
        ],
    )(h, batch2, wn, bn, wp, bp)


def kernel(x, edge_index, batch, W1, b1, W2, b2, eps, Wn, bn, Wp, bp):
    n_layers = W1.shape[0]
    t = Wp.shape[1]
    src = edge_index[0].astype(jnp.int32).reshape(NW, CH, C)
    dst = edge_index[1].astype(jnp.int32).reshape(NW, CH, C)
    batch2 = batch.astype(jnp.int32).reshape(N, 1)
    zeros = jnp.zeros((N, D), ADT)

    hb = x.astype(ADT)
    for i in range(n_layers):
        parts = _agg(hb, src, dst, zeros)
        scale = (1.0 + eps[i]).reshape(1).astype(jnp.float32)
        hb = _gin_mlp(scale, hb, parts[0], parts[1], W1[i],
                      b1[i].reshape(1, D), W2[i], b2[i].reshape(1, D))
    return _head(hb, batch2, Wn, bn.reshape(1, D), Wp, bp.reshape(1, t))
